# baseline (device time: 101021 ns/iter reference)
import jax
import jax.numpy as jnp
from jax import lax
from jax.experimental import pallas as pl
from jax.experimental.pallas import tpu as pltpu

N_CHUNKS = 4


def kernel(x, pi):
    shard_shape = x.shape
    rows = shard_shape[1]
    chunk = rows // N_CHUNKS

    def body(x_ref, pi_ref, out_ref, send_sems, recv_sems, copy_sem):
        my_x = lax.axis_index("x")
        my_y = lax.axis_index("y")
        dst_y = jnp.where(my_y == 0, pi_ref[0], pi_ref[1])
        barrier_sem = pltpu.get_barrier_semaphore()

        @pl.when(dst_y == my_y)
        def _local():
            cp = pltpu.make_async_copy(x_ref, out_ref, copy_sem)
            cp.start()
            cp.wait()

        @pl.when(dst_y != my_y)
        def _swap():
            pl.semaphore_signal(
                barrier_sem,
                inc=1,
                device_id=(my_x, dst_y),
                device_id_type=pl.DeviceIdType.MESH,
            )
            pl.semaphore_wait(barrier_sem, 1)
            rdmas = []
            for c in range(N_CHUNKS):
                rdma = pltpu.make_async_remote_copy(
                    src_ref=x_ref.at[:, pl.ds(c * chunk, chunk), :],
                    dst_ref=out_ref.at[:, pl.ds(c * chunk, chunk), :],
                    send_sem=send_sems.at[c],
                    recv_sem=recv_sems.at[c],
                    device_id=(my_x, dst_y),
                    device_id_type=pl.DeviceIdType.MESH,
                )
                rdma.start()
                rdmas.append(rdma)
            for rdma in rdmas:
                rdma.wait()

    return pl.pallas_call(
        body,
        out_shape=jax.ShapeDtypeStruct(shard_shape, x.dtype),
        in_specs=[
            pl.BlockSpec(memory_space=pl.ANY),
            pl.BlockSpec(memory_space=pltpu.SMEM),
        ],
        out_specs=pl.BlockSpec(memory_space=pl.ANY),
        scratch_shapes=[
            pltpu.SemaphoreType.DMA((N_CHUNKS,)),
            pltpu.SemaphoreType.DMA((N_CHUNKS,)),
            pltpu.SemaphoreType.DMA,
        ],
        compiler_params=pltpu.CompilerParams(collective_id=0),
    )(x, pi)
